# baseline (speedup 1.0000x reference)
import functools

import jax
import jax.numpy as jnp
from jax import lax
from jax.experimental import pallas as pl
from jax.experimental.pallas import tpu as pltpu
from jax.experimental.pallas import tpu_sc as plsc

N = 4096
BATCH = 64
NW = 32
NSLICE = 1
SROWS = N // NSLICE
ROWS_PER_W = SROWS // NW
K = 512
GSUB = 128
NQ = BATCH // 16
JBLK = 1024


def _sc_spmv(xT, cols, rows_arr, vals, starts, nnz_pad, sl_idx):
    mesh = plsc.VectorSubcoreMesh(core_axis_name="c", subcore_axis_name="s")
    maxbase = nnz_pad - K

    @functools.partial(
        pl.kernel,
        out_type=jax.ShapeDtypeStruct((SROWS, BATCH), jnp.float32),
        mesh=mesh,
        scratch_types=[
            pltpu.VMEM((K,), jnp.int32),
            pltpu.VMEM((K,), jnp.int32),
            pltpu.VMEM((K,), jnp.int32),
            pltpu.VMEM((K,), jnp.int32),
            pltpu.VMEM((K,), jnp.float32),
            pltpu.VMEM((K,), jnp.float32),
            pltpu.VMEM((K, BATCH), jnp.float32),
            pltpu.VMEM((K, BATCH), jnp.float32),
            pltpu.VMEM((ROWS_PER_W, BATCH), jnp.float32),
            pltpu.VMEM((144,), jnp.int32),
            pltpu.VMEM_SHARED((N, BATCH), jnp.float32),
            pltpu.SemaphoreType.DMA,
            pltpu.SemaphoreType.DMA,
            pltpu.SemaphoreType.DMA,
            pltpu.SemaphoreType.DMA,
            pltpu.SemaphoreType.DMA,
        ],
        compiler_params=pltpu.CompilerParams(use_tc_tiling_on_sc=False),
    )
    def k(xT_hbm, cols_hbm, rows_hbm, vals_hbm, starts_hbm, out_hbm,
          colv0, colv1, rowv0, rowv1, valv0, valv1, gath0, gath1,
          acc, startsv, xsh, semX, semL0, semL1, semG0, semG1):
        wid = lax.axis_index("s") * 2 + lax.axis_index("c")
        gw = sl_idx * NW + wid
        row_base = gw * ROWS_PER_W

        pltpu.sync_copy(starts_hbm, startsv)
        svec = startsv[pl.ds(gw, 16)]
        s0 = svec[0]
        s1 = svec[1]
        base0 = (s0 // 8) * 8

        def cbase(c):
            return pl.multiple_of(jnp.minimum(base0 + c * K, maxbase), 8)

        def issue_lin(c, colv, rowv, valv, semL):
            b = cbase(c)
            pltpu.async_copy(cols_hbm.at[pl.ds(b, K)], colv, semL)
            pltpu.async_copy(rows_hbm.at[pl.ds(b, K)], rowv, semL)
            pltpu.async_copy(vals_hbm.at[pl.ds(b, K)], valv, semL)

        def wait_lin(colv, rowv, valv, semL):
            pltpu.make_async_copy(cols_hbm.at[pl.ds(0, K)], colv, semL).wait()
            pltpu.make_async_copy(rows_hbm.at[pl.ds(0, K)], rowv, semL).wait()
            pltpu.make_async_copy(vals_hbm.at[pl.ds(0, K)], valv, semL).wait()

        def issue_gath(colv, gath, semG):
            for g in range(K // GSUB):
                pltpu.async_copy(
                    xsh.at[colv.at[pl.ds(g * GSUB, GSUB)]],
                    gath.at[pl.ds(g * GSUB, GSUB)], semG)

        def wait_gath(colv, gath, semG):
            for g in range(K // GSUB):
                pltpu.make_async_copy(
                    xsh.at[colv.at[pl.ds(g * GSUB, GSUB)]],
                    gath.at[pl.ds(g * GSUB, GSUB)], semG).wait()

        def compute(c, rowv, valv, gath):
            b = cbase(c)
            jlo = jnp.maximum(s0 - b, 0)
            jhi = jnp.minimum(s1 - b, K)

            @plsc.parallel_loop(jlo // 16, (jhi + 15) // 16, unroll=2)
            def gbody(g):
                jb = g * 16
                rows16v = rowv[pl.ds(jb, 16)] - row_base
                vals16v = valv[pl.ds(jb, 16)]
                jidx = jb + lax.iota(jnp.int32, 16)
                inr = (jidx >= jlo) & (jidx < jhi)
                rows16 = jnp.clip(rows16v, 0, ROWS_PER_W - 1)
                vals16 = jnp.where(inr, vals16v,
                                   jnp.zeros((16,), jnp.float32))
                for t in range(16):
                    r = rows16[t]
                    v = vals16[t]
                    for q in range(NQ):
                        sl = pl.ds(q * 16, 16)
                        plsc.addupdate(acc.at[r, sl],
                                       v * gath[jb + t, sl])

        sid = lax.axis_index("s")
        xrows = N // 16
        xcp = pltpu.async_copy(xT_hbm.at[pl.ds(sid * xrows, xrows)],
                               xsh.at[pl.ds(sid * xrows, xrows)], semX)
        issue_lin(0, colv0, rowv0, valv0, semL0)
        issue_lin(1, colv1, rowv1, valv1, semL1)

        def zbody(r, _):
            for q in range(NQ):
                acc[r, pl.ds(q * 16, 16)] = jnp.zeros((16,), jnp.float32)
            return 0
        lax.fori_loop(0, ROWS_PER_W, zbody, 0)

        xcp.wait()
        plsc.subcore_barrier()

        wait_lin(colv0, rowv0, valv0, semL0)
        issue_gath(colv0, gath0, semG0)

        nchunks = (s1 - base0 + (K - 1)) // K
        npairs = (nchunks + 1) // 2

        def pair_body(i, _):
            a = 2 * i
            wait_gath(colv0, gath0, semG0)
            wait_lin(colv1, rowv1, valv1, semL1)
            issue_gath(colv1, gath1, semG1)
            compute(a, rowv0, valv0, gath0)
            issue_lin(a + 2, colv0, rowv0, valv0, semL0)
            wait_lin(colv0, rowv0, valv0, semL0)
            issue_gath(colv0, gath0, semG0)
            wait_gath(colv1, gath1, semG1)
            compute(a + 1, rowv1, valv1, gath1)
            issue_lin(a + 3, colv1, rowv1, valv1, semL1)
            return 0
        lax.fori_loop(0, npairs, pair_body, 0)

        wait_gath(colv0, gath0, semG0)
        wait_lin(colv1, rowv1, valv1, semL1)

        pltpu.sync_copy(acc, out_hbm.at[pl.ds(wid * ROWS_PER_W, ROWS_PER_W)])

    return k(xT, cols, rows_arr, vals, starts)


def _tc_partial(p, W, bc_i, flag_i, prev, bc, flag, sl_idx, last):
    def body(*refs):
        if prev is None:
            p_ref, bci_ref, fi_ref, w_ref, *rest = refs
            acc = 0.0
        else:
            p_ref, bci_ref, fi_ref, w_ref, acc_ref, *rest = refs
            acc = acc_ref[...]
        if last:
            bcj_ref, fj_ref, o_ref = rest
        else:
            (o_ref,) = rest
        h2 = bci_ref[...] + p_ref[...] * fi_ref[...]
        o = acc + lax.dot_general(h2, w_ref[...],
                                  (((0,), (1,)), ((), ())),
                                  preferred_element_type=jnp.float32)
        if last:
            o_ref[...] = bcj_ref[...] + o * fj_ref[...]
        else:
            o_ref[...] = o

    in_specs = [
        pl.BlockSpec((SROWS, BATCH), lambda j: (0, 0)),
        pl.BlockSpec((SROWS, 1), lambda j: (0, 0)),
        pl.BlockSpec((SROWS, 1), lambda j: (0, 0)),
        pl.BlockSpec((JBLK, SROWS), lambda j: (j, sl_idx)),
    ]
    args = [p, bc_i.reshape(SROWS, 1), flag_i.reshape(SROWS, 1), W]
    if prev is not None:
        in_specs.append(pl.BlockSpec((BATCH, JBLK), lambda j: (0, j)))
        args.append(prev)
    if last:
        in_specs += [pl.BlockSpec((1, JBLK), lambda j: (0, j)),
                     pl.BlockSpec((1, JBLK), lambda j: (0, j))]
        args += [bc.reshape(1, N), flag.reshape(1, N)]

    return pl.pallas_call(
        body,
        grid=(N // JBLK,),
        in_specs=in_specs,
        out_specs=pl.BlockSpec((BATCH, JBLK), lambda j: (0, j)),
        out_shape=jax.ShapeDtypeStruct((BATCH, N), jnp.float32),
    )(*args)


def kernel(x, B_indices, B_values, bc_value, interior_flag, W):
    rows = B_indices[0]
    cols = B_indices[1]
    nnz = B_values.shape[0]
    nnz_pad = ((nnz + 2 * K + 7) // 8) * 8 + 8
    pad = nnz_pad - nnz
    cols_p = jnp.pad(cols, (0, pad))
    rows_p = jnp.pad(rows, (0, pad))
    vals_p = jnp.pad(B_values, (0, pad))
    nworkers = NSLICE * NW
    bounds = jnp.arange(0, nworkers + 1, dtype=jnp.int32) * ROWS_PER_W
    starts = jnp.searchsorted(rows, bounds, side="left").astype(jnp.int32)
    starts = jnp.pad(starts, (0, 144 - (nworkers + 1)))
    xT = x.T

    out = None
    for s in range(NSLICE):
        p = _sc_spmv(xT, cols_p, rows_p, vals_p, starts, nnz_pad, s)
        lo = s * SROWS
        out = _tc_partial(p, W, bc_value[lo:lo + SROWS],
                          interior_flag[lo:lo + SROWS], out,
                          bc_value, interior_flag, s, s == NSLICE - 1)
    return out


# speedup vs baseline: 1.0062x; 1.0062x over previous
import functools

import jax
import jax.numpy as jnp
from jax import lax
from jax.experimental import pallas as pl
from jax.experimental.pallas import tpu as pltpu
from jax.experimental.pallas import tpu_sc as plsc

N = 4096
BATCH = 64
NW = 32
NSLICE = 1
SROWS = N // NSLICE
ROWS_PER_W = SROWS // NW
K = 512
GSUB = 256
NQ = BATCH // 16
JBLK = 512


def _sc_spmv(xT, cols, rows_arr, vals, starts, nnz_pad, sl_idx):
    mesh = plsc.VectorSubcoreMesh(core_axis_name="c", subcore_axis_name="s")
    maxbase = nnz_pad - K

    @functools.partial(
        pl.kernel,
        out_type=jax.ShapeDtypeStruct((SROWS, BATCH), jnp.float32),
        mesh=mesh,
        scratch_types=[
            pltpu.VMEM((K,), jnp.int32),
            pltpu.VMEM((K,), jnp.int32),
            pltpu.VMEM((K,), jnp.int32),
            pltpu.VMEM((K,), jnp.int32),
            pltpu.VMEM((K,), jnp.float32),
            pltpu.VMEM((K,), jnp.float32),
            pltpu.VMEM((K, BATCH), jnp.float32),
            pltpu.VMEM((K, BATCH), jnp.float32),
            pltpu.VMEM((ROWS_PER_W, BATCH), jnp.float32),
            pltpu.VMEM((144,), jnp.int32),
            pltpu.VMEM_SHARED((N, BATCH), jnp.float32),
            pltpu.SemaphoreType.DMA,
            pltpu.SemaphoreType.DMA,
            pltpu.SemaphoreType.DMA,
            pltpu.SemaphoreType.DMA,
            pltpu.SemaphoreType.DMA,
        ],
        compiler_params=pltpu.CompilerParams(use_tc_tiling_on_sc=False),
    )
    def k(xT_hbm, cols_hbm, rows_hbm, vals_hbm, starts_hbm, out_hbm,
          colv0, colv1, rowv0, rowv1, valv0, valv1, gath0, gath1,
          acc, startsv, xsh, semX, semL0, semL1, semG0, semG1):
        wid = lax.axis_index("s") * 2 + lax.axis_index("c")
        gw = sl_idx * NW + wid
        row_base = gw * ROWS_PER_W

        pltpu.sync_copy(starts_hbm, startsv)
        svec = startsv[pl.ds(gw, 16)]
        s0 = svec[0]
        s1 = svec[1]
        base0 = (s0 // 8) * 8

        def cbase(c):
            return pl.multiple_of(jnp.minimum(base0 + c * K, maxbase), 8)

        def issue_lin(c, colv, rowv, valv, semL):
            b = cbase(c)
            pltpu.async_copy(cols_hbm.at[pl.ds(b, K)], colv, semL)
            pltpu.async_copy(rows_hbm.at[pl.ds(b, K)], rowv, semL)
            pltpu.async_copy(vals_hbm.at[pl.ds(b, K)], valv, semL)

        def wait_lin(colv, rowv, valv, semL):
            pltpu.make_async_copy(cols_hbm.at[pl.ds(0, K)], colv, semL).wait()
            pltpu.make_async_copy(rows_hbm.at[pl.ds(0, K)], rowv, semL).wait()
            pltpu.make_async_copy(vals_hbm.at[pl.ds(0, K)], valv, semL).wait()

        def issue_gath(colv, gath, semG):
            for g in range(K // GSUB):
                pltpu.async_copy(
                    xsh.at[colv.at[pl.ds(g * GSUB, GSUB)]],
                    gath.at[pl.ds(g * GSUB, GSUB)], semG)

        def wait_gath(colv, gath, semG):
            for g in range(K // GSUB):
                pltpu.make_async_copy(
                    xsh.at[colv.at[pl.ds(g * GSUB, GSUB)]],
                    gath.at[pl.ds(g * GSUB, GSUB)], semG).wait()

        def compute(c, rowv, valv, gath):
            b = cbase(c)
            jlo = jnp.maximum(s0 - b, 0)
            jhi = jnp.minimum(s1 - b, K)

            @plsc.parallel_loop(jlo // 16, (jhi + 15) // 16, unroll=2)
            def gbody(g):
                jb = g * 16
                rows16v = rowv[pl.ds(jb, 16)] - row_base
                vals16v = valv[pl.ds(jb, 16)]
                jidx = jb + lax.iota(jnp.int32, 16)
                inr = (jidx >= jlo) & (jidx < jhi)
                rows16 = jnp.clip(rows16v, 0, ROWS_PER_W - 1)
                vals16 = jnp.where(inr, vals16v,
                                   jnp.zeros((16,), jnp.float32))
                for t in range(16):
                    r = rows16[t]
                    v = vals16[t]
                    for q in range(NQ):
                        sl = pl.ds(q * 16, 16)
                        plsc.addupdate(acc.at[r, sl],
                                       v * gath[jb + t, sl])

        sid = lax.axis_index("s")
        xrows = N // 16
        xcp = pltpu.async_copy(xT_hbm.at[pl.ds(sid * xrows, xrows)],
                               xsh.at[pl.ds(sid * xrows, xrows)], semX)
        issue_lin(0, colv0, rowv0, valv0, semL0)
        issue_lin(1, colv1, rowv1, valv1, semL1)

        def zbody(r, _):
            for q in range(NQ):
                acc[r, pl.ds(q * 16, 16)] = jnp.zeros((16,), jnp.float32)
            return 0
        lax.fori_loop(0, ROWS_PER_W, zbody, 0)

        xcp.wait()
        plsc.subcore_barrier()

        wait_lin(colv0, rowv0, valv0, semL0)
        issue_gath(colv0, gath0, semG0)

        nchunks = (s1 - base0 + (K - 1)) // K
        npairs = (nchunks + 1) // 2

        def pair_body(i, _):
            a = 2 * i
            wait_gath(colv0, gath0, semG0)
            wait_lin(colv1, rowv1, valv1, semL1)
            issue_gath(colv1, gath1, semG1)
            compute(a, rowv0, valv0, gath0)
            issue_lin(a + 2, colv0, rowv0, valv0, semL0)
            wait_lin(colv0, rowv0, valv0, semL0)
            issue_gath(colv0, gath0, semG0)
            wait_gath(colv1, gath1, semG1)
            compute(a + 1, rowv1, valv1, gath1)
            issue_lin(a + 3, colv1, rowv1, valv1, semL1)
            return 0
        lax.fori_loop(0, npairs, pair_body, 0)

        wait_gath(colv0, gath0, semG0)
        wait_lin(colv1, rowv1, valv1, semL1)

        pltpu.sync_copy(acc, out_hbm.at[pl.ds(wid * ROWS_PER_W, ROWS_PER_W)])

    return k(xT, cols, rows_arr, vals, starts)


def _tc_partial(p, W, bc_i, flag_i, prev, bc, flag, sl_idx, last):
    def body(*refs):
        if prev is None:
            p_ref, bci_ref, fi_ref, w_ref, *rest = refs
            acc = 0.0
        else:
            p_ref, bci_ref, fi_ref, w_ref, acc_ref, *rest = refs
            acc = acc_ref[...]
        if last:
            bcj_ref, fj_ref, o_ref = rest
        else:
            (o_ref,) = rest
        h2 = bci_ref[...] + p_ref[...] * fi_ref[...]
        o = acc + lax.dot_general(h2, w_ref[...],
                                  (((0,), (1,)), ((), ())),
                                  preferred_element_type=jnp.float32)
        if last:
            o_ref[...] = bcj_ref[...] + o * fj_ref[...]
        else:
            o_ref[...] = o

    in_specs = [
        pl.BlockSpec((SROWS, BATCH), lambda j: (0, 0)),
        pl.BlockSpec((SROWS, 1), lambda j: (0, 0)),
        pl.BlockSpec((SROWS, 1), lambda j: (0, 0)),
        pl.BlockSpec((JBLK, SROWS), lambda j: (j, sl_idx)),
    ]
    args = [p, bc_i.reshape(SROWS, 1), flag_i.reshape(SROWS, 1), W]
    if prev is not None:
        in_specs.append(pl.BlockSpec((BATCH, JBLK), lambda j: (0, j)))
        args.append(prev)
    if last:
        in_specs += [pl.BlockSpec((1, JBLK), lambda j: (0, j)),
                     pl.BlockSpec((1, JBLK), lambda j: (0, j))]
        args += [bc.reshape(1, N), flag.reshape(1, N)]

    return pl.pallas_call(
        body,
        grid=(N // JBLK,),
        in_specs=in_specs,
        out_specs=pl.BlockSpec((BATCH, JBLK), lambda j: (0, j)),
        out_shape=jax.ShapeDtypeStruct((BATCH, N), jnp.float32),
    )(*args)


def kernel(x, B_indices, B_values, bc_value, interior_flag, W):
    rows = B_indices[0]
    cols = B_indices[1]
    nnz = B_values.shape[0]
    nnz_pad = ((nnz + 2 * K + 7) // 8) * 8 + 8
    pad = nnz_pad - nnz
    cols_p = jnp.pad(cols, (0, pad))
    rows_p = jnp.pad(rows, (0, pad))
    vals_p = jnp.pad(B_values, (0, pad))
    nworkers = NSLICE * NW
    bounds = jnp.arange(0, nworkers + 1, dtype=jnp.int32) * ROWS_PER_W
    starts = jnp.searchsorted(rows, bounds, side="left").astype(jnp.int32)
    starts = jnp.pad(starts, (0, 144 - (nworkers + 1)))
    xT = x.T

    out = None
    for s in range(NSLICE):
        p = _sc_spmv(xT, cols_p, rows_p, vals_p, starts, nnz_pad, s)
        lo = s * SROWS
        out = _tc_partial(p, W, bc_value[lo:lo + SROWS],
                          interior_flag[lo:lo + SROWS], out,
                          bc_value, interior_flag, s, s == NSLICE - 1)
    return out


# speedup vs baseline: 1.0489x; 1.0424x over previous
import functools

import jax
import jax.numpy as jnp
from jax import lax
from jax.experimental import pallas as pl
from jax.experimental.pallas import tpu as pltpu
from jax.experimental.pallas import tpu_sc as plsc

N = 4096
BATCH = 64
NW = 32
NSLICE = 1
SROWS = N // NSLICE
ROWS_PER_W = SROWS // NW
K = 768
GSUB = 256
NQ = BATCH // 16
JBLK = 512


def _sc_spmv(xT, cols, rows_arr, vals, starts, nnz_pad, sl_idx):
    mesh = plsc.VectorSubcoreMesh(core_axis_name="c", subcore_axis_name="s")
    maxbase = nnz_pad - K

    @functools.partial(
        pl.kernel,
        out_type=jax.ShapeDtypeStruct((SROWS, BATCH), jnp.float32),
        mesh=mesh,
        scratch_types=[
            pltpu.VMEM((K,), jnp.int32),
            pltpu.VMEM((K,), jnp.int32),
            pltpu.VMEM((K,), jnp.int32),
            pltpu.VMEM((K,), jnp.int32),
            pltpu.VMEM((K,), jnp.float32),
            pltpu.VMEM((K,), jnp.float32),
            pltpu.VMEM((K, BATCH), jnp.float32),
            pltpu.VMEM((K, BATCH), jnp.float32),
            pltpu.VMEM((ROWS_PER_W, BATCH), jnp.float32),
            pltpu.VMEM((144,), jnp.int32),
            pltpu.VMEM_SHARED((N, BATCH), jnp.float32),
            pltpu.SemaphoreType.DMA,
            pltpu.SemaphoreType.DMA,
            pltpu.SemaphoreType.DMA,
            pltpu.SemaphoreType.DMA,
            pltpu.SemaphoreType.DMA,
        ],
        compiler_params=pltpu.CompilerParams(use_tc_tiling_on_sc=False),
    )
    def k(xT_hbm, cols_hbm, rows_hbm, vals_hbm, starts_hbm, out_hbm,
          colv0, colv1, rowv0, rowv1, valv0, valv1, gath0, gath1,
          acc, startsv, xsh, semX, semL0, semL1, semG0, semG1):
        wid = lax.axis_index("s") * 2 + lax.axis_index("c")
        gw = sl_idx * NW + wid
        row_base = gw * ROWS_PER_W

        pltpu.sync_copy(starts_hbm, startsv)
        svec = startsv[pl.ds(gw, 16)]
        s0 = svec[0]
        s1 = svec[1]
        base0 = (s0 // 8) * 8

        def cbase(c):
            return pl.multiple_of(jnp.minimum(base0 + c * K, maxbase), 8)

        def issue_lin(c, colv, rowv, valv, semL):
            b = cbase(c)
            pltpu.async_copy(cols_hbm.at[pl.ds(b, K)], colv, semL)
            pltpu.async_copy(rows_hbm.at[pl.ds(b, K)], rowv, semL)
            pltpu.async_copy(vals_hbm.at[pl.ds(b, K)], valv, semL)

        def wait_lin(colv, rowv, valv, semL):
            pltpu.make_async_copy(cols_hbm.at[pl.ds(0, K)], colv, semL).wait()
            pltpu.make_async_copy(rows_hbm.at[pl.ds(0, K)], rowv, semL).wait()
            pltpu.make_async_copy(vals_hbm.at[pl.ds(0, K)], valv, semL).wait()

        def issue_gath(colv, gath, semG):
            for g in range(K // GSUB):
                pltpu.async_copy(
                    xsh.at[colv.at[pl.ds(g * GSUB, GSUB)]],
                    gath.at[pl.ds(g * GSUB, GSUB)], semG)

        def wait_gath(colv, gath, semG):
            for g in range(K // GSUB):
                pltpu.make_async_copy(
                    xsh.at[colv.at[pl.ds(g * GSUB, GSUB)]],
                    gath.at[pl.ds(g * GSUB, GSUB)], semG).wait()

        def compute(c, rowv, valv, gath):
            b = cbase(c)
            jlo = jnp.maximum(s0 - b, 0)
            jhi = jnp.minimum(s1 - b, K)

            @plsc.parallel_loop(jlo // 16, (jhi + 15) // 16, unroll=2)
            def gbody(g):
                jb = g * 16
                rows16v = rowv[pl.ds(jb, 16)] - row_base
                vals16v = valv[pl.ds(jb, 16)]
                jidx = jb + lax.iota(jnp.int32, 16)
                inr = (jidx >= jlo) & (jidx < jhi)
                rows16 = jnp.clip(rows16v, 0, ROWS_PER_W - 1)
                vals16 = jnp.where(inr, vals16v,
                                   jnp.zeros((16,), jnp.float32))
                for t in range(16):
                    r = rows16[t]
                    v = vals16[t]
                    for q in range(NQ):
                        sl = pl.ds(q * 16, 16)
                        plsc.addupdate(acc.at[r, sl],
                                       v * gath[jb + t, sl])

        sid = lax.axis_index("s")
        xrows = N // 16
        xcp = pltpu.async_copy(xT_hbm.at[pl.ds(sid * xrows, xrows)],
                               xsh.at[pl.ds(sid * xrows, xrows)], semX)
        issue_lin(0, colv0, rowv0, valv0, semL0)
        issue_lin(1, colv1, rowv1, valv1, semL1)

        def zbody(r, _):
            for q in range(NQ):
                acc[r, pl.ds(q * 16, 16)] = jnp.zeros((16,), jnp.float32)
            return 0
        lax.fori_loop(0, ROWS_PER_W, zbody, 0)

        xcp.wait()
        plsc.subcore_barrier()

        wait_lin(colv0, rowv0, valv0, semL0)
        issue_gath(colv0, gath0, semG0)

        nchunks = (s1 - base0 + (K - 1)) // K
        npairs = (nchunks + 1) // 2

        def pair_body(i, _):
            a = 2 * i
            wait_gath(colv0, gath0, semG0)
            wait_lin(colv1, rowv1, valv1, semL1)
            issue_gath(colv1, gath1, semG1)
            compute(a, rowv0, valv0, gath0)
            issue_lin(a + 2, colv0, rowv0, valv0, semL0)
            wait_lin(colv0, rowv0, valv0, semL0)
            issue_gath(colv0, gath0, semG0)
            wait_gath(colv1, gath1, semG1)
            compute(a + 1, rowv1, valv1, gath1)
            issue_lin(a + 3, colv1, rowv1, valv1, semL1)
            return 0
        lax.fori_loop(0, npairs, pair_body, 0)

        wait_gath(colv0, gath0, semG0)
        wait_lin(colv1, rowv1, valv1, semL1)

        pltpu.sync_copy(acc, out_hbm.at[pl.ds(wid * ROWS_PER_W, ROWS_PER_W)])

    return k(xT, cols, rows_arr, vals, starts)


def _tc_partial(p, W, bc_i, flag_i, prev, bc, flag, sl_idx, last):
    def body(*refs):
        if prev is None:
            p_ref, bci_ref, fi_ref, w_ref, *rest = refs
            acc = 0.0
        else:
            p_ref, bci_ref, fi_ref, w_ref, acc_ref, *rest = refs
            acc = acc_ref[...]
        if last:
            bcj_ref, fj_ref, o_ref = rest
        else:
            (o_ref,) = rest
        h2 = bci_ref[...] + p_ref[...] * fi_ref[...]
        o = acc + lax.dot_general(h2, w_ref[...],
                                  (((0,), (1,)), ((), ())),
                                  preferred_element_type=jnp.float32)
        if last:
            o_ref[...] = bcj_ref[...] + o * fj_ref[...]
        else:
            o_ref[...] = o

    in_specs = [
        pl.BlockSpec((SROWS, BATCH), lambda j: (0, 0)),
        pl.BlockSpec((SROWS, 1), lambda j: (0, 0)),
        pl.BlockSpec((SROWS, 1), lambda j: (0, 0)),
        pl.BlockSpec((JBLK, SROWS), lambda j: (j, sl_idx)),
    ]
    args = [p, bc_i.reshape(SROWS, 1), flag_i.reshape(SROWS, 1), W]
    if prev is not None:
        in_specs.append(pl.BlockSpec((BATCH, JBLK), lambda j: (0, j)))
        args.append(prev)
    if last:
        in_specs += [pl.BlockSpec((1, JBLK), lambda j: (0, j)),
                     pl.BlockSpec((1, JBLK), lambda j: (0, j))]
        args += [bc.reshape(1, N), flag.reshape(1, N)]

    return pl.pallas_call(
        body,
        grid=(N // JBLK,),
        in_specs=in_specs,
        out_specs=pl.BlockSpec((BATCH, JBLK), lambda j: (0, j)),
        out_shape=jax.ShapeDtypeStruct((BATCH, N), jnp.float32),
    )(*args)


def kernel(x, B_indices, B_values, bc_value, interior_flag, W):
    rows = B_indices[0]
    cols = B_indices[1]
    nnz = B_values.shape[0]
    nnz_pad = ((nnz + 2 * K + 7) // 8) * 8 + 8
    pad = nnz_pad - nnz
    cols_p = jnp.pad(cols, (0, pad))
    rows_p = jnp.pad(rows, (0, pad))
    vals_p = jnp.pad(B_values, (0, pad))
    nworkers = NSLICE * NW
    bounds = jnp.arange(0, nworkers + 1, dtype=jnp.int32) * ROWS_PER_W
    starts = jnp.searchsorted(rows, bounds, side="left").astype(jnp.int32)
    starts = jnp.pad(starts, (0, 144 - (nworkers + 1)))
    xT = x.T

    out = None
    for s in range(NSLICE):
        p = _sc_spmv(xT, cols_p, rows_p, vals_p, starts, nnz_pad, s)
        lo = s * SROWS
        out = _tc_partial(p, W, bc_value[lo:lo + SROWS],
                          interior_flag[lo:lo + SROWS], out,
                          bc_value, interior_flag, s, s == NSLICE - 1)
    return out
